# baseline (speedup 1.0000x reference)
import functools

import jax
import jax.numpy as jnp
from jax import lax
from jax.experimental import pallas as pl
from jax.experimental.pallas import tpu as pltpu
from jax.experimental.pallas import tpu_sc as plsc

_B, _H, _W = 16, 512, 512
_K = 128
_RB = 512
_FP = 64.0

_NW = 32
_SPLIT = 2
_BS = _B // _SPLIT
_SROWS = _B * _H // _SPLIT
_WROWS = _SROWS // _NW
_CR = 32
_NCHUNK = _WROWS // _CR


def _make_sc_body(base):
    def _sc_gather_body(tbl_h, inst_h, gp_h, tb_v, in_b, out_b,
                        sem_i, sem_o):
        c = lax.axis_index("c")
        s = lax.axis_index("s")
        wid = s * 2 + c
        b = (base + wid * _WROWS) // _H
        pltpu.sync_copy(tbl_h.at[pl.ds(b * _K, _K)], tb_v)
        row0 = base + wid * _WROWS
        out0 = wid * _WROWS

        def in_copy(ci, par):
            return pltpu.make_async_copy(
                inst_h.at[pl.ds(row0 + ci * _CR, _CR)],
                in_b.at[pl.ds(par * _CR, _CR)], sem_i.at[par])

        def out_copy(ci, par):
            return pltpu.make_async_copy(
                out_b.at[pl.ds(par * _CR, _CR)],
                gp_h.at[pl.ds(out0 + ci * _CR, _CR)], sem_o.at[par])

        in_copy(0, 0).start()

        def chunk_body(ci, carry):
            par = lax.rem(ci, 2)
            nxt = 1 - par
            in_copy(ci, par).wait()

            @pl.when(ci + 1 < _NCHUNK)
            def _():
                in_copy(ci + 1, nxt).start()

            @pl.when(ci >= 2)
            def _():
                out_copy(ci - 2, par).wait()

            @plsc.parallel_loop(0, _CR, unroll=2)
            def row_body(r):
                rr = par * _CR + r
                for g in range(_W // 16):
                    cc = g * 16
                    idx = in_b[rr, pl.ds(cc, 16)]
                    out_b[rr, pl.ds(cc, 16)] = plsc.load_gather(tb_v, [idx])

            out_copy(ci, par).start()
            return carry

        lax.fori_loop(0, _NCHUNK, chunk_body, 0)
        out_copy(_NCHUNK - 2, 0).wait()
        out_copy(_NCHUNK - 1, 1).wait()

    return _sc_gather_body


def _sc_gather(tbl, inst, half):
    mesh = plsc.VectorSubcoreMesh(core_axis_name="c", subcore_axis_name="s")
    f = pl.kernel(
        _make_sc_body(half * _SROWS),
        out_type=jax.ShapeDtypeStruct((_SROWS, _W), jnp.int32),
        mesh=mesh,
        compiler_params=pltpu.CompilerParams(needs_layout_passes=False),
        scratch_types=(
            pltpu.VMEM((_K,), jnp.int32),
            pltpu.VMEM((2 * _CR, _W), jnp.int32),
            pltpu.VMEM((2 * _CR, _W), jnp.int32),
            pltpu.SemaphoreType.DMA((2,)),
            pltpu.SemaphoreType.DMA((2,)),
        ),
    )
    return f(tbl, inst)


def _fast_atan2(y, x):
    ax = jnp.abs(x)
    ay = jnp.abs(y)
    mx = jnp.maximum(ax, ay)
    t = jnp.minimum(ax, ay) / jnp.maximum(mx, 1e-30)
    s = t * t
    p = t * (0.99921406 + s * (-0.32117747 + s * (0.14627053 + s * (-0.03899059))))
    p = jnp.where(ay > ax, 1.5707963267948966 - p, p)
    p = jnp.where(x < 0.0, 3.141592653589793 - p, p)
    return jnp.where(y < 0.0, -p, p)


def _tc_compute(gp_ref, out_ref):
    w = gp_ref[0]
    mask = w >= 0
    mf = mask.astype(jnp.float32)
    cy = jnp.where(mask, (w >> 16).astype(jnp.float32) * (1.0 / _FP),
                   -10000.0)
    cx = jnp.where(mask, (w & 0xFFFF).astype(jnp.float32) * (1.0 / _FP),
                   -10000.0)
    row = lax.broadcasted_iota(jnp.int32, (_RB, _W), 0).astype(jnp.float32)
    col = lax.broadcasted_iota(jnp.int32, (_RB, _W), 1).astype(jnp.float32)
    x = cx - row
    y = cy - col
    cmask = 1.0 - ((jnp.abs(x) < 3.0) & (jnp.abs(y) < 3.0)).astype(jnp.float32)
    r2 = x * x + y * y
    inv = lax.rsqrt(jnp.maximum(r2, 1e-12))
    minv = mf * inv
    out_ref[0, 0] = r2 * minv
    out_ref[0, 1] = _fast_atan2(y, x)
    out_ref[0, 2] = y * minv
    out_ref[0, 3] = x * minv
    out_ref[0, 4] = cmask


def _tc_body_first(gp_ref, out_ref):
    _tc_compute(gp_ref, out_ref)


def _tc_body_second(prev_ref, gp_ref, out_ref):
    del prev_ref
    _tc_compute(gp_ref, out_ref)


@functools.partial(jax.jit, static_argnames=())
def kernel(instances, centers, batch_index):
    del batch_index
    inst = instances.reshape(_B * _H, _W)
    yq = jnp.round(centers[:, :, 0] * _FP).astype(jnp.int32)
    xq = jnp.round(centers[:, :, 1] * _FP).astype(jnp.int32)
    packed = (yq << 16) | xq
    neg = jnp.full((_B, 1), -1, jnp.int32)
    pad = jnp.full((_B, _K - 101), -1, jnp.int32)
    tbl = jnp.concatenate([neg, packed, pad], axis=1).reshape(-1)

    gp0 = _sc_gather(tbl, inst, 0).reshape(_BS, _H, _W)
    gp1 = _sc_gather(tbl, inst, 1).reshape(_BS, _H, _W)

    out_shape = jax.ShapeDtypeStruct((_B, 5, _H, _W), jnp.float32)
    out0 = pl.pallas_call(
        _tc_body_first,
        grid=(_BS,),
        in_specs=[pl.BlockSpec((1, _RB, _W), lambda b: (b, 0, 0))],
        out_specs=pl.BlockSpec((1, 5, _RB, _W), lambda b: (b, 0, 0, 0)),
        out_shape=out_shape,
    )(gp0)
    out = pl.pallas_call(
        _tc_body_second,
        grid=(_BS,),
        in_specs=[
            pl.BlockSpec(memory_space=pl.ANY),
            pl.BlockSpec((1, _RB, _W), lambda b: (b, 0, 0)),
        ],
        out_specs=pl.BlockSpec((1, 5, _RB, _W), lambda b: (b + _BS, 0, 0, 0)),
        out_shape=out_shape,
        input_output_aliases={0: 0},
    )(out0, gp1)
    return out


# speedup vs baseline: 1.0051x; 1.0051x over previous
import functools

import jax
import jax.numpy as jnp
from jax import lax
from jax.experimental import pallas as pl
from jax.experimental.pallas import tpu as pltpu
from jax.experimental.pallas import tpu_sc as plsc

_B, _H, _W = 16, 512, 512
_K = 128
_RB = 512
_FP = 64.0

_NW = 32
_HROWS = _B * _H // _NW
_CR = 32
_NCHUNK = _HROWS // _CR


def _sc_gather_body(tbl_h, inst_h, gp_h, tb_v, in_b, out_b,
                    sem_i, sem_o):
    c = lax.axis_index("c")
    s = lax.axis_index("s")
    wid = s * 2 + c
    b = wid // 2
    pltpu.sync_copy(tbl_h.at[pl.ds(b * _K, _K)], tb_v)
    row0 = wid * _HROWS

    def in_copy(ci, par):
        return pltpu.make_async_copy(
            inst_h.at[pl.ds(row0 + ci * _CR, _CR)],
            in_b.at[pl.ds(par * _CR, _CR)], sem_i.at[par])

    def out_copy(ci, par):
        return pltpu.make_async_copy(
            out_b.at[pl.ds(par * _CR, _CR)],
            gp_h.at[pl.ds(row0 + ci * _CR, _CR)], sem_o.at[par])

    in_copy(0, 0).start()

    def chunk_body(ci, carry):
        par = lax.rem(ci, 2)
        nxt = 1 - par
        in_copy(ci, par).wait()

        @pl.when(ci + 1 < _NCHUNK)
        def _():
            in_copy(ci + 1, nxt).start()

        @pl.when(ci >= 2)
        def _():
            out_copy(ci - 2, par).wait()

        @plsc.parallel_loop(0, _CR, unroll=2)
        def row_body(r):
            rr = par * _CR + r
            for g in range(_W // 16):
                cc = g * 16
                idx = in_b[rr, pl.ds(cc, 16)]
                out_b[rr, pl.ds(cc, 16)] = plsc.load_gather(tb_v, [idx])

        out_copy(ci, par).start()
        return carry

    lax.fori_loop(0, _NCHUNK, chunk_body, 0)
    out_copy(_NCHUNK - 2, 0).wait()
    out_copy(_NCHUNK - 1, 1).wait()


def _sc_gather(tbl, inst):
    mesh = plsc.VectorSubcoreMesh(core_axis_name="c", subcore_axis_name="s")
    f = pl.kernel(
        _sc_gather_body,
        out_type=jax.ShapeDtypeStruct((_B * _H, _W), jnp.int32),
        mesh=mesh,
        compiler_params=pltpu.CompilerParams(needs_layout_passes=False),
        scratch_types=(
            pltpu.VMEM((_K,), jnp.int32),
            pltpu.VMEM((2 * _CR, _W), jnp.int32),
            pltpu.VMEM((2 * _CR, _W), jnp.int32),
            pltpu.SemaphoreType.DMA((2,)),
            pltpu.SemaphoreType.DMA((2,)),
        ),
    )
    return f(tbl, inst)


def _fast_atan2(y, x):
    ax = jnp.abs(x)
    ay = jnp.abs(y)
    mx = jnp.maximum(ax, ay)
    t = jnp.minimum(ax, ay) / jnp.maximum(mx, 1e-30)
    s = t * t
    p = t * (0.99921406 + s * (-0.32117747 + s * (0.14627053 + s * (-0.03899059))))
    p = jnp.where(ay > ax, 1.5707963267948966 - p, p)
    p = jnp.where(x < 0.0, 3.141592653589793 - p, p)
    return jnp.where(y < 0.0, -p, p)


def _tc_body(gp_ref, out_ref):
    j = pl.program_id(1)
    w = gp_ref[0]
    mask = w >= 0
    mf = mask.astype(jnp.float32)
    cy = jnp.where(mask, (w >> 16).astype(jnp.float32) * (1.0 / _FP),
                   -10000.0)
    cx = jnp.where(mask, (w & 0xFFFF).astype(jnp.float32) * (1.0 / _FP),
                   -10000.0)
    row = (j * _RB + lax.broadcasted_iota(jnp.int32, (_RB, _W), 0)
           ).astype(jnp.float32)
    col = lax.broadcasted_iota(jnp.int32, (_RB, _W), 1).astype(jnp.float32)
    x = cx - row
    y = cy - col
    cmask = 1.0 - ((jnp.abs(x) < 3.0) & (jnp.abs(y) < 3.0)).astype(jnp.float32)
    r2 = x * x + y * y
    inv = lax.rsqrt(jnp.maximum(r2, 1e-12))
    minv = mf * inv
    out_ref[0, 0] = r2 * minv
    out_ref[0, 1] = _fast_atan2(y, x)
    out_ref[0, 2] = y * minv
    out_ref[0, 3] = x * minv
    out_ref[0, 4] = cmask


@functools.partial(jax.jit, static_argnames=())
def kernel(instances, centers, batch_index):
    del batch_index
    inst = instances.reshape(_B * _H, _W)
    yq = jnp.round(centers[:, :, 0] * _FP).astype(jnp.int32)
    xq = jnp.round(centers[:, :, 1] * _FP).astype(jnp.int32)
    packed = (yq << 16) | xq
    neg = jnp.full((_B, 1), -1, jnp.int32)
    pad = jnp.full((_B, _K - 101), -1, jnp.int32)
    tbl = jnp.concatenate([neg, packed, pad], axis=1).reshape(-1)

    gp = _sc_gather(tbl, inst).reshape(_B, _H, _W)

    out = pl.pallas_call(
        _tc_body,
        grid=(_B, _H // _RB),
        in_specs=[
            pl.BlockSpec((1, _RB, _W), lambda b, j: (b, j, 0)),
        ],
        out_specs=pl.BlockSpec((1, 5, _RB, _W), lambda b, j: (b, 0, j, 0)),
        out_shape=jax.ShapeDtypeStruct((_B, 5, _H, _W), jnp.float32),
    )(gp)
    return out
